# baseline (speedup 1.0000x reference)
import functools

import jax
import jax.numpy as jnp
from jax import lax
from jax.experimental import pallas as pl
from jax.experimental.pallas import tpu as pltpu
from jax.experimental.pallas import tpu_sc as plsc

B_TOTAL = 16384
D = 256
N_CB = 8192

M_TILE = 1024
N_CHUNK = 2048

ST_TILE = 2048

_DIMNUMS_LAST = (((1,), (1,)), ((), ()))


def _scan_tile(z_ref, cb_ref, cn_ref, out_ref):
    z_t = z_ref[...]
    zn = jnp.sum(z_t * z_t, axis=1, keepdims=True)
    z_m2 = z_t * jnp.float32(-2.0)
    io128 = lax.broadcasted_iota(jnp.int32, (M_TILE, 128), 1).astype(jnp.float32)
    v_run = jnp.full((M_TILE, 128), jnp.inf, jnp.float32)
    j_run = jnp.zeros((M_TILE, 128), jnp.float32)
    def _chunk_dot(c):
        cb_c = cb_ref[pl.ds(c * N_CHUNK, N_CHUNK), :]
        return lax.dot_general(z_m2, cb_c, _DIMNUMS_LAST,
                               preferred_element_type=jnp.float32)

    n_chunks = N_CB // N_CHUNK
    s2_next = _chunk_dot(0)
    for c in range(n_chunks):
        s2 = s2_next
        if c + 1 < n_chunks:
            s2_next = _chunk_dot(c + 1)
        for jj in range(N_CHUNK // 128):
            col = jj * 128
            d_sl = (zn + cn_ref[:, pl.ds(c * N_CHUNK + col, 128)]) + s2[:, col:col + 128]
            m = d_sl < v_run
            v_run = jnp.where(m, d_sl, v_run)
            j_run = jnp.where(m, jnp.float32(c * (N_CHUNK // 128) + jj), j_run)
    gmin = jnp.min(v_run, axis=1, keepdims=True)
    idxf = j_run * jnp.float32(128.0) + io128
    cand = jnp.where(v_run == gmin, idxf, jnp.float32(N_CB))
    out_ref[...] = jnp.min(cand, axis=1).astype(jnp.int32)


B_HALF = B_TOTAL // 2


def _argmin_body_a(z_ref, e_ref, w_ref, idx_ref, cb_out_ref, cn_out_ref,
                   cb_ref, cn_ref):
    @pl.when(pl.program_id(0) == 0)
    def _():
        cb = lax.dot_general(e_ref[...], w_ref[...], _DIMNUMS_LAST,
                             preferred_element_type=jnp.float32)
        cb_ref[...] = cb
        cb_out_ref[...] = cb
        cn = lax.dot_general(jnp.ones((1, D), jnp.float32), cb * cb,
                             _DIMNUMS_LAST, preferred_element_type=jnp.float32)
        cn_ref[...] = cn
        cn_out_ref[...] = cn

    _scan_tile(z_ref, cb_ref, cn_ref, idx_ref)


def _argmin_body_b(z_ref, cb_ref, cn_ref, idx_ref):
    _scan_tile(z_ref, cb_ref, cn_ref, idx_ref)


def _compute_indices_a(z_flat, embedding_weight, proj_weight):
    return pl.pallas_call(
        _argmin_body_a,
        grid=(B_HALF // M_TILE,),
        in_specs=[
            pl.BlockSpec((M_TILE, D), lambda i: (i, 0)),
            pl.BlockSpec((N_CB, D), lambda i: (0, 0)),
            pl.BlockSpec((D, D), lambda i: (0, 0)),
        ],
        out_specs=(
            pl.BlockSpec((M_TILE,), lambda i: (i,)),
            pl.BlockSpec((N_CB, D), lambda i: (0, 0)),
            pl.BlockSpec((1, N_CB), lambda i: (0, 0)),
        ),
        out_shape=(
            jax.ShapeDtypeStruct((B_HALF,), jnp.int32),
            jax.ShapeDtypeStruct((N_CB, D), jnp.float32),
            jax.ShapeDtypeStruct((1, N_CB), jnp.float32),
        ),
        scratch_shapes=[
            pltpu.VMEM((N_CB, D), jnp.float32),
            pltpu.VMEM((1, N_CB), jnp.float32),
        ],
    )(z_flat, embedding_weight, proj_weight)


def _compute_indices_b(z_flat, codebook, cnorm_row):
    half_blocks = B_HALF // M_TILE
    return pl.pallas_call(
        _argmin_body_b,
        grid=(half_blocks,),
        in_specs=[
            pl.BlockSpec((M_TILE, D), lambda i: (i + half_blocks, 0)),
            pl.BlockSpec((N_CB, D), lambda i: (0, 0)),
            pl.BlockSpec((1, N_CB), lambda i: (0, 0)),
        ],
        out_specs=pl.BlockSpec((M_TILE,), lambda i: (i,)),
        out_shape=jax.ShapeDtypeStruct((B_HALF,), jnp.int32),
    )(z_flat, codebook, cnorm_row)


_NUM_SC_CORES = 2
_NUM_SC_SUBCORES = 16
_NW = _NUM_SC_CORES * _NUM_SC_SUBCORES
_ROWS_PER_W = B_HALF // _NW
_GCHUNK = 128


def _gather_rows(codebook, indices):
    mesh = plsc.VectorSubcoreMesh(core_axis_name="c", subcore_axis_name="s")
    n_chunks = _ROWS_PER_W // _GCHUNK

    @functools.partial(
        pl.kernel, mesh=mesh,
        out_type=jax.ShapeDtypeStruct((B_HALF, D), jnp.float32),
        scratch_types=(
            [pltpu.VMEM((_GCHUNK,), jnp.int32) for _ in range(3)]
            + [pltpu.VMEM((_GCHUNK, D), jnp.float32) for _ in range(3)]
            + [pltpu.SemaphoreType.DMA for _ in range(6)]
        ),
    )
    def gather_k(table_hbm, idx_hbm, out_hbm, i0, i1, i2, r0, r1, r2,
                 g0, g1, g2, s0, s1, s2):
        idxs, rows = [i0, i1, i2], [r0, r1, r2]
        gsems, ssems = [g0, g1, g2], [s0, s1, s2]
        wid = lax.axis_index("s") * _NUM_SC_CORES + lax.axis_index("c")
        base = wid * _ROWS_PER_W
        gh = [None] * n_chunks
        sh = [None] * n_chunks
        for c in range(min(3, n_chunks)):
            off = base + c * _GCHUNK
            pltpu.sync_copy(idx_hbm.at[pl.ds(off, _GCHUNK)], idxs[c])
            gh[c] = pltpu.async_copy(table_hbm.at[idxs[c]], rows[c], gsems[c])
        for c in range(n_chunks):
            b = c % 3
            gh[c].wait()
            sh[c] = pltpu.async_copy(
                rows[b], out_hbm.at[pl.ds(base + c * _GCHUNK, _GCHUNK)], ssems[b])
            nxt = c + 3
            if nxt < n_chunks:
                sh[c].wait()
                off = base + nxt * _GCHUNK
                pltpu.sync_copy(idx_hbm.at[pl.ds(off, _GCHUNK)], idxs[b])
                gh[nxt] = pltpu.async_copy(table_hbm.at[idxs[b]], rows[b], gsems[b])
        for c in range(n_chunks):
            if c + 3 >= n_chunks:
                sh[c].wait()

    return gather_k(codebook, indices)


def _st_body(zq_ref, z_ref, out_ref, acc_ref):
    i = pl.program_id(0)
    zq = zq_ref[...]
    zt = z_ref[...]
    diff = zq - zt
    out_ref[...] = zt + diff
    s = jnp.sum(diff * diff)

    @pl.when(i == 0)
    def _():
        acc_ref[0, 0] = s

    @pl.when(i > 0)
    def _():
        acc_ref[0, 0] = acc_ref[0, 0] + s


def _straight_through(z_q_flat, z_flat):
    return pl.pallas_call(
        _st_body,
        grid=(B_TOTAL // ST_TILE,),
        in_specs=[
            pl.BlockSpec((ST_TILE, D), lambda i: (i, 0)),
            pl.BlockSpec((ST_TILE, D), lambda i: (i, 0)),
        ],
        out_specs=(
            pl.BlockSpec((ST_TILE, D), lambda i: (i, 0)),
            pl.BlockSpec(memory_space=pltpu.SMEM),
        ),
        out_shape=(
            jax.ShapeDtypeStruct((B_TOTAL, D), jnp.float32),
            jax.ShapeDtypeStruct((1, 1), jnp.float32),
        ),
    )(z_q_flat, z_flat)


def kernel(z, embedding_weight, proj_weight):
    b, c, h, w = z.shape
    z_t = jnp.transpose(z, (0, 2, 3, 1))
    z_flat = z_t.reshape(-1, c)

    idx1, codebook, cnorm_row = _compute_indices_a(
        z_flat, embedding_weight, proj_weight)
    zq1 = _gather_rows(codebook, idx1)
    idx2 = _compute_indices_b(z_flat, codebook, cnorm_row)
    zq2 = _gather_rows(codebook, idx2)
    indices = jnp.concatenate([idx1, idx2], axis=0)
    z_q_flat = jnp.concatenate([zq1, zq2], axis=0)
    z_q_st, sq_sum = _straight_through(z_q_flat, z_flat)
    z_q_out = jnp.transpose(z_q_st.reshape(b, h, w, c), (0, 3, 1, 2))

    m = sq_sum[0, 0] / jnp.float32(B_TOTAL * D)
    commitment_loss = jnp.float32(0.25) * m
    codebook_loss = m
    loss = commitment_loss + codebook_loss

    indices_out = indices.reshape(b, h, w)
    return (z_q_out, loss, commitment_loss, codebook_loss, indices_out)


# speedup vs baseline: 1.1310x; 1.1310x over previous
import functools

import jax
import jax.numpy as jnp
from jax import lax
from jax.experimental import pallas as pl
from jax.experimental.pallas import tpu as pltpu
from jax.experimental.pallas import tpu_sc as plsc

B_TOTAL = 16384
D = 256
N_CB = 8192

M_TILE = 1024
N_CHUNK = 2048

ST_TILE = 2048

_DIMNUMS_LAST = (((1,), (1,)), ((), ()))


def _scan_tile(z_ref, cb_ref, cn_ref, out_ref):
    z_t = z_ref[...]
    zn = jnp.sum(z_t * z_t, axis=1, keepdims=True)
    z_m2 = z_t * jnp.float32(-2.0)
    io128 = lax.broadcasted_iota(jnp.int32, (M_TILE, 128), 1).astype(jnp.float32)
    v_run = jnp.full((M_TILE, 128), jnp.inf, jnp.float32)
    j_run = jnp.zeros((M_TILE, 128), jnp.float32)
    def _chunk_dot(c):
        cb_c = cb_ref[pl.ds(c * N_CHUNK, N_CHUNK), :]
        return lax.dot_general(z_m2, cb_c, _DIMNUMS_LAST,
                               preferred_element_type=jnp.float32)

    n_chunks = N_CB // N_CHUNK
    s2_next = _chunk_dot(0)
    for c in range(n_chunks):
        s2 = s2_next
        if c + 1 < n_chunks:
            s2_next = _chunk_dot(c + 1)
        for jj in range(N_CHUNK // 128):
            col = jj * 128
            d_sl = (zn + cn_ref[:, pl.ds(c * N_CHUNK + col, 128)]) + s2[:, col:col + 128]
            m = d_sl < v_run
            v_run = jnp.where(m, d_sl, v_run)
            j_run = jnp.where(m, jnp.float32(c * (N_CHUNK // 128) + jj), j_run)
    gmin = jnp.min(v_run, axis=1, keepdims=True)
    idxf = j_run * jnp.float32(128.0) + io128
    cand = jnp.where(v_run == gmin, idxf, jnp.float32(N_CB))
    out_ref[...] = jnp.min(cand, axis=1).astype(jnp.int32)


def _argmin_body(z_ref, e_ref, w_ref, idx_ref, cb_out_ref, cb_ref, cn_ref):
    @pl.when(pl.program_id(0) == 0)
    def _():
        cb = lax.dot_general(e_ref[...], w_ref[...], _DIMNUMS_LAST,
                             preferred_element_type=jnp.float32)
        cb_ref[...] = cb
        cb_out_ref[...] = cb
        cn_ref[...] = lax.dot_general(jnp.ones((1, D), jnp.float32), cb * cb,
                                      _DIMNUMS_LAST,
                                      preferred_element_type=jnp.float32)

    _scan_tile(z_ref, cb_ref, cn_ref, idx_ref)


def _compute_indices(z_flat, embedding_weight, proj_weight):
    return pl.pallas_call(
        _argmin_body,
        grid=(B_TOTAL // M_TILE,),
        in_specs=[
            pl.BlockSpec((M_TILE, D), lambda i: (i, 0)),
            pl.BlockSpec((N_CB, D), lambda i: (0, 0)),
            pl.BlockSpec((D, D), lambda i: (0, 0)),
        ],
        out_specs=(
            pl.BlockSpec((M_TILE,), lambda i: (i,)),
            pl.BlockSpec((N_CB, D), lambda i: (0, 0)),
        ),
        out_shape=(
            jax.ShapeDtypeStruct((B_TOTAL,), jnp.int32),
            jax.ShapeDtypeStruct((N_CB, D), jnp.float32),
        ),
        scratch_shapes=[
            pltpu.VMEM((N_CB, D), jnp.float32),
            pltpu.VMEM((1, N_CB), jnp.float32),
        ],
    )(z_flat, embedding_weight, proj_weight)


_NUM_SC_CORES = 2
_NUM_SC_SUBCORES = 16
_NW = _NUM_SC_CORES * _NUM_SC_SUBCORES
_ROWS_PER_W = B_TOTAL // _NW
_GCHUNK = 128


def _gather_rows(codebook, indices):
    mesh = plsc.VectorSubcoreMesh(core_axis_name="c", subcore_axis_name="s")
    n_chunks = _ROWS_PER_W // _GCHUNK

    @functools.partial(
        pl.kernel, mesh=mesh,
        out_type=jax.ShapeDtypeStruct((B_TOTAL, D), jnp.float32),
        scratch_types=(
            [pltpu.VMEM((_GCHUNK,), jnp.int32) for _ in range(3)]
            + [pltpu.VMEM((_GCHUNK, D), jnp.float32) for _ in range(3)]
            + [pltpu.SemaphoreType.DMA for _ in range(6)]
        ),
    )
    def gather_k(table_hbm, idx_hbm, out_hbm, i0, i1, i2, r0, r1, r2,
                 g0, g1, g2, s0, s1, s2):
        idxs, rows = [i0, i1, i2], [r0, r1, r2]
        gsems, ssems = [g0, g1, g2], [s0, s1, s2]
        wid = lax.axis_index("s") * _NUM_SC_CORES + lax.axis_index("c")
        base = wid * _ROWS_PER_W
        gh = [None] * n_chunks
        sh = [None] * n_chunks
        for c in range(min(3, n_chunks)):
            off = base + c * _GCHUNK
            pltpu.sync_copy(idx_hbm.at[pl.ds(off, _GCHUNK)], idxs[c])
            gh[c] = pltpu.async_copy(table_hbm.at[idxs[c]], rows[c], gsems[c])
        for c in range(n_chunks):
            b = c % 3
            gh[c].wait()
            sh[c] = pltpu.async_copy(
                rows[b], out_hbm.at[pl.ds(base + c * _GCHUNK, _GCHUNK)], ssems[b])
            nxt = c + 3
            if nxt < n_chunks:
                sh[c].wait()
                off = base + nxt * _GCHUNK
                pltpu.sync_copy(idx_hbm.at[pl.ds(off, _GCHUNK)], idxs[b])
                gh[nxt] = pltpu.async_copy(table_hbm.at[idxs[b]], rows[b], gsems[b])
        for c in range(n_chunks):
            if c + 3 >= n_chunks:
                sh[c].wait()

    return gather_k(codebook, indices)


def _st_body(zq_ref, z_ref, out_ref, acc_ref):
    i = pl.program_id(0)
    zq = zq_ref[...]
    zt = z_ref[...]
    diff = zq - zt
    out_ref[...] = zt + diff
    s = jnp.sum(diff * diff)

    @pl.when(i == 0)
    def _():
        acc_ref[0, 0] = s

    @pl.when(i > 0)
    def _():
        acc_ref[0, 0] = acc_ref[0, 0] + s


def _straight_through(z_q_flat, z_flat):
    return pl.pallas_call(
        _st_body,
        grid=(B_TOTAL // ST_TILE,),
        in_specs=[
            pl.BlockSpec((ST_TILE, D), lambda i: (i, 0)),
            pl.BlockSpec((ST_TILE, D), lambda i: (i, 0)),
        ],
        out_specs=(
            pl.BlockSpec((ST_TILE, D), lambda i: (i, 0)),
            pl.BlockSpec(memory_space=pltpu.SMEM),
        ),
        out_shape=(
            jax.ShapeDtypeStruct((B_TOTAL, D), jnp.float32),
            jax.ShapeDtypeStruct((1, 1), jnp.float32),
        ),
    )(z_q_flat, z_flat)


def kernel(z, embedding_weight, proj_weight):
    b, c, h, w = z.shape
    z_t = jnp.transpose(z, (0, 2, 3, 1))
    z_flat = z_t.reshape(-1, c)

    indices, codebook = _compute_indices(z_flat, embedding_weight, proj_weight)
    z_q_flat = _gather_rows(codebook, indices)
    z_q_st, sq_sum = _straight_through(z_q_flat, z_flat)
    z_q_out = jnp.transpose(z_q_st.reshape(b, h, w, c), (0, 3, 1, 2))

    m = sq_sum[0, 0] / jnp.float32(B_TOTAL * D)
    commitment_loss = jnp.float32(0.25) * m
    codebook_loss = m
    loss = commitment_loss + codebook_loss

    indices_out = indices.reshape(b, h, w)
    return (z_q_out, loss, commitment_loss, codebook_loss, indices_out)
